# Initial kernel scaffold; baseline (speedup 1.0000x reference)
import jax
import jax.numpy as jnp
from jax.experimental import pallas as pl


def kernel(x, up_index, down_index):
    raise NotImplementedError("write your pallas kernel here")


# speedup vs baseline: 3.4703x; 3.4703x over previous
import functools

import jax
import jax.numpy as jnp
from jax import lax
from jax.experimental import pallas as pl
from jax.experimental.pallas import tpu as pltpu
from jax.experimental.pallas import tpu_sc as plsc

N_NODES = 10000
D_FEAT = 256
HALF = D_FEAT // 2
NUM_SC = 2
NUM_TEC = 16
CHUNK = 128

ACC_ROWS = 10112
ZERO_ROWS = ACC_ROWS // NUM_TEC
OUT_ROWS = 624
TAIL_ROWS = N_NODES - NUM_TEC * OUT_ROWS


def _sc_kernel(e_pad, n_chunks):
    per_tile = n_chunks * CHUNK

    def body(xs_hbm, src_hbm, dst_hbm, zer_hbm, out_hbm,
             src_v, dst_v, rows_v, acc, sem):
        c = lax.axis_index("c")
        s = lax.axis_index("s")
        base = c * e_pad + s * per_tile

        pltpu.sync_copy(zer_hbm, acc.at[pl.ds(s * ZERO_ROWS, ZERO_ROWS)])
        plsc.subcore_barrier()

        def chunk(g, carry):
            e0 = base + g * CHUNK
            pltpu.sync_copy(src_hbm.at[pl.ds(e0, CHUNK)], src_v)
            pltpu.sync_copy(dst_hbm.at[pl.ds(e0, CHUNK)], dst_v)
            pltpu.async_copy(xs_hbm.at[src_v], rows_v, sem).wait()
            pltpu.sync_copy(rows_v, acc.at[dst_v], add=True)
            return carry

        lax.fori_loop(0, n_chunks, chunk, 0)
        plsc.subcore_barrier()

        pltpu.sync_copy(
            acc.at[pl.ds(s * OUT_ROWS, OUT_ROWS)],
            out_hbm.at[pl.ds(s * OUT_ROWS, OUT_ROWS), pl.ds(c * HALF, HALF)])

        @pl.when(s == NUM_TEC - 1)
        def _tail():
            r0 = NUM_TEC * OUT_ROWS
            pltpu.sync_copy(
                acc.at[pl.ds(r0, TAIL_ROWS)],
                out_hbm.at[pl.ds(r0, TAIL_ROWS), pl.ds(c * HALF, HALF)])

    mesh = plsc.VectorSubcoreMesh(core_axis_name="c", subcore_axis_name="s")
    return pl.kernel(
        body,
        out_type=jax.ShapeDtypeStruct((N_NODES, D_FEAT), jnp.float32),
        mesh=mesh,
        scratch_types=[
            pltpu.VMEM((CHUNK,), jnp.int32),
            pltpu.VMEM((CHUNK,), jnp.int32),
            pltpu.VMEM((CHUNK, HALF), jnp.float32),
            pltpu.VMEM_SHARED((ACC_ROWS, HALF), jnp.float32),
            pltpu.SemaphoreType.DMA,
        ],
    )


@jax.jit
def kernel(x, up_index, down_index):
    n_edges = up_index.shape[1] + down_index.shape[1]
    align = NUM_TEC * CHUNK
    e_pad = ((n_edges + align - 1) // align) * align
    n_chunks = e_pad // align
    pad = e_pad - n_edges

    src = jnp.concatenate(
        [up_index[0], down_index[0], jnp.zeros((pad,), up_index.dtype)]
    ).astype(jnp.int32)
    dst = jnp.concatenate(
        [up_index[1], down_index[1],
         jnp.full((pad,), N_NODES, up_index.dtype)]
    ).astype(jnp.int32)
    src_all = jnp.concatenate([src, src + N_NODES])
    dst_all = jnp.concatenate([dst, dst])
    xs = jnp.concatenate([x[:, :HALF], x[:, HALF:]], axis=0)
    zer = jnp.zeros((ZERO_ROWS, HALF), jnp.float32)

    return _sc_kernel(e_pad, n_chunks)(xs, src_all, dst_all, zer)
